# Initial kernel scaffold; baseline (speedup 1.0000x reference)
import jax
import jax.numpy as jnp
from jax.experimental import pallas as pl


def kernel(points_xyz, features, rois, w1, b1, g1, be1, w2, b2, g2, be2):
    raise NotImplementedError("write your pallas kernel here")


# speedup vs baseline: 1.0013x; 1.0013x over previous
import jax
import jax.numpy as jnp
from jax.experimental import pallas as pl

_OUT = 5


def _conv_bn_relu(x, w, b, g, be):
    x = jnp.einsum('oc,bcn->bon', w, x) + b[None, :, None]
    mean = jnp.mean(x, axis=(0, 2), keepdims=True)
    var = jnp.var(x, axis=(0, 2), keepdims=True)
    x = (x - mean) / jnp.sqrt(var + 1e-5) * g[None, :, None] + be[None, :, None]
    return jax.nn.relu(x)


def _roi_pool(points, feats_t, rois):
    out = _OUT

    def pool_one(pts, f, roi):
        center = roi[:3]
        dims = jnp.maximum(roi[3:6], 1e-6)
        yaw = roi[6]
        c, s = jnp.cos(yaw), jnp.sin(yaw)
        local = pts - center[None, :]
        lx = local[:, 0] * c + local[:, 1] * s
        ly = -local[:, 0] * s + local[:, 1] * c
        lz = local[:, 2]
        vx = jnp.floor((lx / dims[0] + 0.5) * out).astype(jnp.int32)
        vy = jnp.floor((ly / dims[1] + 0.5) * out).astype(jnp.int32)
        vz = jnp.floor((lz / dims[2] + 0.5) * out).astype(jnp.int32)
        inside = (vx >= 0) & (vx < out) & (vy >= 0) & (vy < out) & (vz >= 0) & (vz < out)
        vid = jnp.where(inside, (vx * out + vy) * out + vz, out ** 3)
        pooled = jax.ops.segment_max(f, vid, num_segments=out ** 3 + 1)[:out ** 3]
        pooled = jnp.where(jnp.isfinite(pooled), pooled, 0.0)
        return pooled.reshape(out, out, out, f.shape[1])

    per_batch = jax.vmap(lambda p, f, rs: jax.vmap(lambda r: pool_one(p, f, r))(rs))
    return per_batch(points, feats_t, rois)


def _identity_kernel(x_ref, o_ref):
    o_ref[...] = x_ref[...]


def kernel(points_xyz, features, rois, w1, b1, g1, be1, w2, b2, g2, be2):
    x = _conv_bn_relu(features, w1, b1, g1, be1)
    x = _conv_bn_relu(x, w2, b2, g2, be2)
    x = pl.pallas_call(
        _identity_kernel,
        out_shape=jax.ShapeDtypeStruct(x.shape, x.dtype),
    )(x)
    roi_feats = _roi_pool(points_xyz, jnp.transpose(x, (0, 2, 1)), rois)
    return (x, roi_feats)


# speedup vs baseline: 5.7436x; 5.7360x over previous
import functools

import jax
import jax.numpy as jnp
from jax import lax
from jax.experimental import pallas as pl
from jax.experimental.pallas import tpu as pltpu
from jax.experimental.pallas import tpu_sc as plsc

_B = 2
_N = 8192
_CIN = 256
_C = 128
_R = 64
_OUT = 5
_NVOX = _OUT ** 3
_PROWS = _NVOX + 1
_PAIRS = _B * _R
_GCHUNK = 128


def _mlp_body(f_ref, w1_ref, b1_ref, g1_ref, be1_ref,
              w2_ref, b2_ref, g2_ref, be2_ref, x_ref, xt_ref):
    def layer(y, g_ref, be_ref):
        mean = jnp.mean(y, axis=1, keepdims=True)
        var = jnp.mean((y - mean) ** 2, axis=1, keepdims=True)
        y = (y - mean) / jnp.sqrt(var + 1e-5) * g_ref[...][:, None] + be_ref[...][:, None]
        return jnp.maximum(y, 0.0)

    f = f_ref[...]
    y1 = jnp.dot(w1_ref[...], f, preferred_element_type=jnp.float32)
    y1 = y1 + b1_ref[...][:, None]
    a1 = layer(y1, g1_ref, be1_ref)
    y2 = jnp.dot(w2_ref[...], a1, preferred_element_type=jnp.float32)
    y2 = y2 + b2_ref[...][:, None]
    x = layer(y2, g2_ref, be2_ref)
    x_ref[...] = x
    xt_ref[...] = x.T


def _mlp(features, w1, b1, g1, be1, w2, b2, g2, be2):
    f = jnp.transpose(features, (1, 0, 2)).reshape(_CIN, _B * _N)
    x, xt = pl.pallas_call(
        _mlp_body,
        out_shape=(
            jax.ShapeDtypeStruct((_C, _B * _N), jnp.float32),
            jax.ShapeDtypeStruct((_B * _N, _C), jnp.float32),
        ),
    )(f, w1, b1, g1, be1, w2, b2, g2, be2)
    return x, xt


def _pool_body(pxyz_hbm, params_hbm, xt_hbm, out_hbm,
               pts_v, par_v, packed_v, idx_v, rows_v, pooled_v, sem):
    wid = lax.axis_index("s") * 2 + lax.axis_index("c")
    b = wid // 16
    pltpu.sync_copy(pxyz_hbm.at[b], pts_v)

    fives = jnp.full((16,), 5.0, jnp.float32)
    zeros16 = jnp.zeros((16,), jnp.float32)
    lane = lax.iota(jnp.int32, 16)

    for k in range(4):
        pair = wid * 4 + k
        r = pair - b * _R
        pltpu.sync_copy(params_hbm.at[b, r], par_v)
        pvec = par_v[...]
        cx = pvec[0]
        cy = pvec[1]
        cz = pvec[2]
        cth = pvec[3]
        sth = pvec[4]
        ddx = pvec[5]
        ddy = pvec[6]
        ddz = pvec[7]

        def zero_row(j, _):
            for c8 in range(8):
                pooled_v[j, pl.ds(c8 * 16, 16)] = zeros16
            return 0
        lax.fori_loop(0, _PROWS, zero_row, 0)

        gbase = b * _N

        def chunk_a(i, off):
            base = i * 16
            px = pts_v[0, pl.ds(base, 16)]
            py = pts_v[1, pl.ds(base, 16)]
            pz = pts_v[2, pl.ds(base, 16)]
            dxv = px - cx
            dyv = py - cy
            lx = dxv * cth + dyv * sth
            ly = dyv * cth - dxv * sth
            lz = pz - cz
            tx = (lx / ddx + 0.5) * 5.0
            ty = (ly / ddy + 0.5) * 5.0
            tz = (lz / ddz + 0.5) * 5.0
            inside = ((tx >= zeros16) & (tx < fives) &
                      (ty >= zeros16) & (ty < fives) &
                      (tz >= zeros16) & (tz < fives))
            vx = tx.astype(jnp.int32)
            vy = ty.astype(jnp.int32)
            vz = tz.astype(jnp.int32)
            vid = (vx * 5 + vy) * 5 + vz
            packed = (vid << 14) | (gbase + base + lane)
            cnt = jnp.sum(inside.astype(jnp.int32))
            plsc.store_compressed(packed_v.at[pl.ds(off, 16)], packed, mask=inside)
            return off + cnt

        m = lax.fori_loop(0, _N // 16, chunk_a, 0)

        dummy = jnp.full((16,), _NVOX << 14, jnp.int32)
        for t in range(_GCHUNK // 16):
            packed_v[pl.ds(m + t * 16, 16)] = dummy

        nchunks = (m + _GCHUNK - 1) // _GCHUNK

        def chunk_b(cix, _):
            base = cix * _GCHUNK
            for t in range(_GCHUNK // 16):
                pk = packed_v[pl.ds(base + t * 16, 16)]
                idx_v[pl.ds(t * 16, 16)] = pk & 0x3FFF
            pltpu.async_copy(xt_hbm.at[idx_v], rows_v, sem).wait()

            def pt(jj, _2):
                vj = packed_v[pl.ds(base + jj, 16)][0] >> 14
                for c8 in range(8):
                    seg = rows_v[jj, pl.ds(c8 * 16, 16)]
                    cur = pooled_v[vj, pl.ds(c8 * 16, 16)]
                    pooled_v[vj, pl.ds(c8 * 16, 16)] = jnp.maximum(cur, seg)
                return 0

            lax.fori_loop(0, _GCHUNK, pt, 0)
            return 0

        lax.fori_loop(0, nchunks, chunk_b, 0)
        pltpu.sync_copy(pooled_v, out_hbm.at[pair])


def _roi_pool_sc(points_xyz, rois, xt):
    pxyz = jnp.transpose(points_xyz, (0, 2, 1))
    center = rois[..., 0:3]
    dims = jnp.maximum(rois[..., 3:6], 1e-6)
    yaw = rois[..., 6:7]
    params = jnp.concatenate(
        [center, jnp.cos(yaw), jnp.sin(yaw), dims,
         jnp.zeros((_B, _R, 8), jnp.float32)], axis=-1)

    mesh = plsc.VectorSubcoreMesh(core_axis_name="c", subcore_axis_name="s")
    pool = pl.kernel(
        _pool_body,
        out_type=jax.ShapeDtypeStruct((_PAIRS, _PROWS, _C), jnp.float32),
        mesh=mesh,
        scratch_types=[
            pltpu.VMEM((3, _N), jnp.float32),
            pltpu.VMEM((16,), jnp.float32),
            pltpu.VMEM((_N + _GCHUNK + 32,), jnp.int32),
            pltpu.VMEM((_GCHUNK,), jnp.int32),
            pltpu.VMEM((_GCHUNK, _C), jnp.float32),
            pltpu.VMEM((_PROWS, _C), jnp.float32),
            pltpu.SemaphoreType.DMA,
        ],
        compiler_params=pltpu.CompilerParams(needs_layout_passes=False),
    )
    out = pool(pxyz, params, xt)
    out = out.reshape(_B, _R, _PROWS, _C)[:, :, :_NVOX, :]
    return out.reshape(_B, _R, _OUT, _OUT, _OUT, _C)


def kernel(points_xyz, features, rois, w1, b1, g1, be1, w2, b2, g2, be2):
    x, xt = _mlp(features, w1, b1, g1, be1, w2, b2, g2, be2)
    roi_feats = _roi_pool_sc(points_xyz, rois, xt)
    x = x.reshape(_C, _B, _N).transpose(1, 0, 2)
    return (x, roi_feats)
